# Initial kernel scaffold; baseline (speedup 1.0000x reference)
import jax
import jax.numpy as jnp
from jax.experimental import pallas as pl


def kernel(x, edge_index, edge_attr, W_msg, b_msg, W_root):
    raise NotImplementedError("write your pallas kernel here")


# speedup vs baseline: 53.4894x; 53.4894x over previous
import functools

import jax
import jax.numpy as jnp
from jax import lax
from jax.experimental import pallas as pl
from jax.experimental.pallas import tpu as pltpu
from jax.experimental.pallas import tpu_sc as plsc

N = 100000
NP = 100096
E = 6400000
OUT = 4
NC, NS, L = 2, 16, 16
NW = NC * NS
ROWS_PER_TILE = NP // NS
SUB = 128
WIN_ROWS = 16
WIN_E = WIN_ROWS * SUB
EROWS = E // SUB
N_WIN = E // WIN_E
WINS_PER_WORKER = (N_WIN + NW - 1) // NW


BL = NP // 2


def _tc_prep_body(xt_ref, wm_ref, bm_ref, wr_ref, gt_ref, baset_ref):
    xt = xt_ref[...]
    wm = wm_ref[...]
    b = bm_ref[...]
    gt_ref[...] = (
        wm[0:1, :].T * xt[0:1, :]
        + wm[1:2, :].T * xt[1:2, :]
        + wm[2:3, :].T * xt[2:3, :]
        + b
    )
    wr = wr_ref[...]
    baset_ref[...] = (
        wr[0:1, :].T * xt[0:1, :]
        + wr[1:2, :].T * xt[1:2, :]
        + wr[2:3, :].T * xt[2:3, :]
    )


def _tc_combine_body(p_ref, baset_ref, out_ref):
    out_ref[...] = p_ref[0] + p_ref[1] + baset_ref[...]


def _sc_edges_body(
    gt_hbm, src_hbm, dst_hbm, e_hbm, we_hbm, z_hbm, out_hbm,
    g0, g1, g2, g3, a0, a1, a2, a3,
    stage, srcv, dstv, ev, c0, c1, c2, c3, wbuf, gsem, ssem,
):
    gs = [g0, g1, g2, g3]
    accs = [a0, a1, a2, a3]
    cols = [c0, c1, c2, c3]
    cid = lax.axis_index("c")
    sid = lax.axis_index("s")
    wid = sid * NC + cid

    r0 = sid * ROWS_PER_TILE
    for d in range(OUT):
        pltpu.sync_copy(gt_hbm.at[pl.ds(d * NP + r0, ROWS_PER_TILE)], stage)
        pltpu.sync_copy(stage, gs[d].at[pl.ds(r0, ROWS_PER_TILE)])
    pltpu.sync_copy(z_hbm.at[pl.ds(r0, ROWS_PER_TILE)], stage)
    for d in range(OUT):
        pltpu.sync_copy(stage, accs[d].at[pl.ds(r0, ROWS_PER_TILE)])
    pltpu.sync_copy(we_hbm, wbuf)
    w_b = [wbuf[d] for d in range(OUT)]
    plsc.subcore_barrier()

    def do_window(t):
        ro = t * WIN_ROWS
        eo = t * WIN_E
        pltpu.sync_copy(src_hbm.at[pl.ds(ro, WIN_ROWS), :], srcv)
        pltpu.sync_copy(dst_hbm.at[pl.ds(ro, WIN_ROWS), :], dstv)
        pltpu.sync_copy(e_hbm.at[pl.ds(eo, WIN_E)], ev)
        descs = []
        for j in range(WIN_ROWS):
            for d in range(OUT):
                descs.append(
                    pltpu.async_copy(
                        gs[d].at[srcv.at[j]],
                        cols[d].at[pl.ds(j * SUB, SUB)],
                        gsem,
                    )
                )
        for dsc in descs:
            dsc.wait()

        def cbody(i, _):
            sl = pl.ds(i * L, L)
            eb = ev[sl]
            for d in range(OUT):
                cols[d][sl] = jnp.maximum(cols[d][sl] + eb * w_b[d], 0.0)
            return 0

        lax.fori_loop(0, WIN_E // L, cbody, 0)

        descs = []
        for j in range(WIN_ROWS):
            for d in range(OUT):
                descs.append(
                    pltpu.async_copy(
                        cols[d].at[pl.ds(j * SUB, SUB)],
                        accs[d].at[dstv.at[j]],
                        ssem,
                        add=True,
                    )
                )
        for dsc in descs:
            dsc.wait()

    def wbody(i, _):
        t = wid + i * NW

        @pl.when(t < N_WIN)
        def _():
            do_window(t)

        return 0

    lax.fori_loop(0, WINS_PER_WORKER, wbody, 0)

    plsc.subcore_barrier()
    for d in range(OUT):
        pltpu.sync_copy(accs[d].at[pl.ds(r0, ROWS_PER_TILE)], stage)
        pltpu.sync_copy(stage, out_hbm.at[pl.ds((cid * OUT + d) * NP + r0, ROWS_PER_TILE)])


_sc_edges = functools.partial(
    pl.kernel,
    out_type=jax.ShapeDtypeStruct((NC * OUT * NP,), jnp.float32),
    mesh=plsc.VectorSubcoreMesh(
        core_axis_name="c", subcore_axis_name="s", num_cores=NC, num_subcores=NS
    ),
    scratch_types=(
        [pltpu.VMEM_SHARED((NP,), jnp.float32) for _ in range(OUT)]
        + [pltpu.VMEM_SHARED((NP,), jnp.float32) for _ in range(OUT)]
        + [
            pltpu.VMEM((ROWS_PER_TILE,), jnp.float32),
            pltpu.VMEM((WIN_ROWS, SUB), jnp.int32),
            pltpu.VMEM((WIN_ROWS, SUB), jnp.int32),
            pltpu.VMEM((WIN_E,), jnp.float32),
        ]
        + [pltpu.VMEM((WIN_E,), jnp.float32) for _ in range(OUT)]
        + [
            pltpu.VMEM((OUT, L), jnp.float32),
            pltpu.SemaphoreType.DMA,
            pltpu.SemaphoreType.DMA,
        ]
    ),
)(_sc_edges_body)


@jax.jit
def kernel(x, edge_index, edge_attr, W_msg, b_msg, W_root):
    src = edge_index[0].astype(jnp.int32).reshape(EROWS, SUB)
    dst = edge_index[1].astype(jnp.int32).reshape(EROWS, SUB)
    e = edge_attr.reshape(E)
    we = jnp.tile(W_msg[3].reshape(OUT, 1), (1, L))
    zeros = jnp.zeros((NP,), jnp.float32)
    xp = jnp.pad(x.T, ((0, 0), (0, NP - N)))

    gt, baset = pl.pallas_call(
        _tc_prep_body,
        grid=(NP // BL,),
        in_specs=[
            pl.BlockSpec((3, BL), lambda i: (0, i)),
            pl.BlockSpec((OUT, OUT), lambda i: (0, 0)),
            pl.BlockSpec((OUT, 1), lambda i: (0, 0)),
            pl.BlockSpec((3, OUT), lambda i: (0, 0)),
        ],
        out_specs=(
            pl.BlockSpec((OUT, BL), lambda i: (0, i)),
            pl.BlockSpec((OUT, BL), lambda i: (0, i)),
        ),
        out_shape=(
            jax.ShapeDtypeStruct((OUT, NP), jnp.float32),
            jax.ShapeDtypeStruct((OUT, NP), jnp.float32),
        ),
    )(xp, W_msg, b_msg.reshape(OUT, 1), W_root)

    p = _sc_edges(gt.reshape(OUT * NP), src, dst, e, we, zeros)
    p = p.reshape(NC, OUT, NP)

    outt = pl.pallas_call(
        _tc_combine_body,
        grid=(NP // BL,),
        in_specs=[
            pl.BlockSpec((NC, OUT, BL), lambda i: (0, 0, i)),
            pl.BlockSpec((OUT, BL), lambda i: (0, i)),
        ],
        out_specs=pl.BlockSpec((OUT, BL), lambda i: (0, i)),
        out_shape=jax.ShapeDtypeStruct((OUT, NP), jnp.float32),
    )(p, baset)
    return outt.T[:N]
